# Initial kernel scaffold; baseline (speedup 1.0000x reference)
import jax
import jax.numpy as jnp
from jax.experimental import pallas as pl


def kernel(gt_boxes, sampled_indices, t, noise):
    raise NotImplementedError("write your pallas kernel here")


# speedup vs baseline: 1.8041x; 1.8041x over previous
import functools

import jax
import jax.numpy as jnp
from jax import lax
from jax.experimental import pallas as pl
from jax.experimental.pallas import tpu as pltpu
from jax.experimental.pallas import tpu_sc as plsc

B = 16
G = 100
P = 500
PP = 512
NW = 32
ITEMS = B * PP
IPW = ITEMS // NW
L = 16
STEPS = IPW // L

_HALF_LOG_A = -0.0010010006671670687


def _sc_body(gt_hbm, idx_hbm, t_hbm, nz_hbm, out_hbm,
             gt_v, idx_v, t_v, nz_v, out_v, sem):
    cid = lax.axis_index("c")
    sid = lax.axis_index("s")
    wid = sid * 2 + cid
    b = wid // 2

    cp_gt = pltpu.async_copy(gt_hbm.at[pl.ds(b * (G * 4), G * 4)], gt_v, sem)
    cp_ix = pltpu.async_copy(idx_hbm.at[pl.ds(wid * IPW, IPW)], idx_v, sem)
    cp_t = pltpu.async_copy(t_hbm.at[pl.ds(wid * IPW, IPW)], t_v, sem)
    cp_nz = pltpu.async_copy(nz_hbm.at[pl.ds(wid * IPW * 4, IPW * 4)], nz_v, sem)
    cp_gt.wait()
    cp_ix.wait()
    cp_t.wait()
    cp_nz.wait()

    lane4 = lax.iota(jnp.int32, 16) * 4
    for i in range(STEPS):
        g = idx_v[pl.ds(i * L, L)]
        tf = t_v[pl.ds(i * L, L)].astype(jnp.float32)
        sa = jnp.exp(tf * _HALF_LOG_A)
        x = 1.0 - sa * sa
        y = lax.bitcast_convert_type(
            0x5F3759DF - (lax.bitcast_convert_type(x, jnp.int32) >> 1),
            jnp.float32)
        for _ in range(3):
            y = y * (1.5 - 0.5 * x * y * y)
        sb = x * y * 1024.0
        gi = g * 4
        for c in range(4):
            nidx = lane4 + (i * L * 4 + c)
            gv = plsc.load_gather(gt_v, [gi + c])
            nv = plsc.load_gather(nz_v, [nidx])
            plsc.store_scatter(out_v, [nidx], gv * sa + nv * sb)

    pltpu.sync_copy(out_v, out_hbm.at[pl.ds(wid * IPW * 4, IPW * 4)])


@jax.jit
def kernel(gt_boxes, sampled_indices, t, noise):
    idx_p = jnp.pad(sampled_indices.astype(jnp.int32),
                    ((0, 0), (0, PP - P))).reshape(-1)
    t_p = jnp.pad(t.astype(jnp.int32), ((0, 0), (0, PP - P))).reshape(-1)
    nz_p = jnp.pad(noise, ((0, 0), (0, PP - P), (0, 0))).reshape(-1)
    gt_flat = gt_boxes.reshape(-1)

    sc = pl.kernel(
        _sc_body,
        out_type=jax.ShapeDtypeStruct((ITEMS * 4,), jnp.float32),
        mesh=plsc.VectorSubcoreMesh(core_axis_name="c", subcore_axis_name="s"),
        compiler_params=pltpu.CompilerParams(needs_layout_passes=False),
        scratch_types=[
            pltpu.VMEM((G * 4,), jnp.float32),
            pltpu.VMEM((IPW,), jnp.int32),
            pltpu.VMEM((IPW,), jnp.int32),
            pltpu.VMEM((IPW * 4,), jnp.float32),
            pltpu.VMEM((IPW * 4,), jnp.float32),
            pltpu.SemaphoreType.DMA,
        ],
    )
    out_flat = sc(gt_flat, idx_p, t_p, nz_p)
    prior = out_flat.reshape(B, PP, 4)[:, :P, :]
    return prior, t, sampled_indices


# speedup vs baseline: 2.5009x; 1.3863x over previous
import jax
import jax.numpy as jnp
from jax import lax
from jax.experimental import pallas as pl
from jax.experimental.pallas import tpu as pltpu
from jax.experimental.pallas import tpu_sc as plsc

B = 16
G = 100
P = 500
L = 16
STEPS = 16
H0_ITEMS = 256
H1_ITEMS = P - H0_ITEMS

_HALF_LOG_A = -0.0010010006671670687


def _sc_body(gt_hbm, idx_hbm, t_hbm, nz_hbm, out_hbm,
             gt_v, idx_v, t_v, nz_v, out_v, sem):
    cid = lax.axis_index("c")
    sid = lax.axis_index("s")
    wid = sid * 2 + cid
    b = wid // 2
    h = wid % 2

    cp_gt = pltpu.async_copy(gt_hbm.at[pl.ds(b * (G * 4), G * 4)], gt_v, sem)
    cp_ix = pltpu.async_copy(idx_hbm.at[b], idx_v.at[pl.ds(0, P)], sem)
    cp_t = pltpu.async_copy(t_hbm.at[b], t_v.at[pl.ds(0, P)], sem)
    cp_nz = pltpu.async_copy(nz_hbm.at[b], nz_v.at[pl.ds(0, P * 4)], sem)
    cp_gt.wait()
    cp_ix.wait()
    cp_t.wait()
    cp_nz.wait()

    lane4 = lax.iota(jnp.int32, 16) * 4
    base = h * H0_ITEMS
    for i in range(STEPS):
        off = base + i * L
        g = idx_v[pl.ds(off, L)]
        tf = t_v[pl.ds(off, L)].astype(jnp.float32)
        g = jnp.minimum(jnp.maximum(g, 0), G - 1)
        sa = jnp.exp(tf * _HALF_LOG_A)
        x = 1.0 - sa * sa
        y = lax.bitcast_convert_type(
            0x5F3759DF - (lax.bitcast_convert_type(x, jnp.int32) >> 1),
            jnp.float32)
        for _ in range(3):
            y = y * (1.5 - 0.5 * x * y * y)
        sb = x * y * 1024.0
        gi = g * 4
        voff = off * 4
        for c in range(4):
            nidx = lane4 + (voff + c)
            gv = plsc.load_gather(gt_v, [gi + c])
            nv = plsc.load_gather(nz_v, [nidx])
            plsc.store_scatter(out_v, [nidx], gv * sa + nv * sb)

    obase = b * (P * 4) + h * (H0_ITEMS * 4)

    @pl.when(h == 0)
    def _():
        pltpu.sync_copy(out_v.at[pl.ds(0, H0_ITEMS * 4)],
                        out_hbm.at[pl.ds(obase, H0_ITEMS * 4)])

    @pl.when(h == 1)
    def _():
        pltpu.sync_copy(out_v.at[pl.ds(H0_ITEMS * 4, H1_ITEMS * 4)],
                        out_hbm.at[pl.ds(obase, H1_ITEMS * 4)])


@jax.jit
def kernel(gt_boxes, sampled_indices, t, noise):
    idx2 = sampled_indices.astype(jnp.int32)
    t2 = t.astype(jnp.int32)
    nz2 = noise.reshape(B, P * 4)
    gt_flat = gt_boxes.reshape(-1)

    sc = pl.kernel(
        _sc_body,
        out_type=jax.ShapeDtypeStruct((B * P * 4,), jnp.float32),
        mesh=plsc.VectorSubcoreMesh(core_axis_name="c", subcore_axis_name="s"),
        compiler_params=pltpu.CompilerParams(needs_layout_passes=False,
                                             use_tc_tiling_on_sc=False),
        scratch_types=[
            pltpu.VMEM((G * 4,), jnp.float32),
            pltpu.VMEM((512,), jnp.int32),
            pltpu.VMEM((512,), jnp.int32),
            pltpu.VMEM((2048,), jnp.float32),
            pltpu.VMEM((2048,), jnp.float32),
            pltpu.SemaphoreType.DMA,
        ],
    )
    out_flat = sc(gt_flat, idx2, t2, nz2)
    prior = out_flat.reshape(B, P, 4)
    return prior, t, sampled_indices
